# baseline (speedup 1.0000x reference)
import functools

import jax
import jax.numpy as jnp
from jax.experimental import pallas as pl
from jax.experimental.pallas import tpu as pltpu
from jax.experimental.pallas import tpu_sc as plsc

N = 4096
D = 2048
FF = 5632
NE = 8
TOPK = 2
T = 256
G = (N * TOPK) // T + NE
NS = G * T
FT = 1408
NF = FF // FT
TB = 256
GB = NS // TB
KC = 2
FH = FF // KC
NC, NSUB = 2, 16
NW = NC * NSUB
CW = 16
NCH = N // (CW * NW)


def _router_body(x_ref, rw_ref, logits_ref):
    logits_ref[...] = jax.lax.dot_general(
        x_ref[...], rw_ref[...], (((1,), (1,)), ((), ())),
        preferred_element_type=jnp.float32)


def _inclusive_prefix(a):
    n = a.shape[0]
    ck = min(512, n)
    row = jax.lax.broadcasted_iota(jnp.int32, (n, ck), 0)
    acc = None
    for k in range(n // ck):
        colk = jax.lax.broadcasted_iota(jnp.int32, (n, ck), 1) + k * ck
        lt = (colk <= row).astype(jnp.float32)
        part = jax.lax.dot_general(
            lt, a[k * ck:(k + 1) * ck, :], (((1,), (0,)), ((), ())),
            preferred_element_type=jnp.float32)
        acc = part if acc is None else acc + part
    return acc


def _plan_body(logits_ref, pos0_ref, pos1_ref, w0_ref, w1_ref,
               be_ref, nb_ref):
    logits = logits_ref[...]
    m = jnp.max(logits, axis=1, keepdims=True)
    ex = jnp.exp(logits - m)
    p = ex / jnp.sum(ex, axis=1, keepdims=True)
    col = jax.lax.broadcasted_iota(jnp.int32, p.shape, 1)
    m1 = jnp.max(p, axis=1, keepdims=True)
    i1 = jnp.min(jnp.where(p == m1, col, NE), axis=1, keepdims=True)
    oh1 = (col == i1).astype(jnp.float32)
    pmask = jnp.where(col == i1, -jnp.inf, p)
    m2 = jnp.max(pmask, axis=1, keepdims=True)
    i2 = jnp.min(jnp.where(pmask == m2, col, NE), axis=1, keepdims=True)
    oh2 = (col == i2).astype(jnp.float32)
    w0_ref[...] = m1
    w1_ref[...] = m2

    counts = jnp.sum(oh1 + oh2, axis=0, keepdims=True)
    nbpe = jnp.floor((counts + (T - 1)) * (1.0 / T))
    r8 = jax.lax.broadcasted_iota(jnp.int32, (NE, NE), 0)
    c8 = jax.lax.broadcasted_iota(jnp.int32, (NE, NE), 1)
    ltm = (r8 < c8).astype(jnp.float32)
    startsb = jax.lax.dot_general(nbpe, ltm, (((1,), (0,)), ((), ())),
                                  preferred_element_type=jnp.float32)
    starts_tok = startsb * float(T)

    s1i = _inclusive_prefix(oh1)
    s2i = _inclusive_prefix(oh2)
    s1x = s1i - oh1
    s2x = s2i - oh2
    rank_a = jnp.sum(oh1 * (s1x + s2x), axis=1, keepdims=True)
    rank_b = jnp.sum(oh2 * (s1i + s2x), axis=1, keepdims=True)
    start_a = jnp.sum(oh1 * starts_tok, axis=1, keepdims=True)
    start_b = jnp.sum(oh2 * starts_tok, axis=1, keepdims=True)
    pos0_ref[...] = (start_a + rank_a).astype(jnp.int32)
    pos1_ref[...] = (start_b + rank_b).astype(jnp.int32)

    sb_colv = jnp.sum((c8 < r8).astype(jnp.float32) * nbpe, axis=1,
                      keepdims=True) * (float(T) / float(TB))
    gi = jax.lax.broadcasted_iota(jnp.int32, (NE, GB), 1).astype(jnp.float32)
    cmp = (sb_colv <= gi).astype(jnp.float32)
    be_ref[...] = (jnp.sum(cmp, axis=0, keepdims=True) - 1.0).astype(jnp.int32)
    nb_ref[...] = jnp.sum(nbpe, axis=1, keepdims=True).astype(jnp.int32)


def _sc_scatter(x, p0m, p1m):
    mesh = plsc.VectorSubcoreMesh(core_axis_name="c", subcore_axis_name="s")

    @functools.partial(
        pl.kernel,
        out_type=jax.ShapeDtypeStruct((NS, D), jnp.float32),
        mesh=mesh,
        scratch_types=[
            pltpu.VMEM((NCH, CW), jnp.int32),
            pltpu.VMEM((NCH, CW), jnp.int32),
            pltpu.VMEM((CW, D), jnp.float32),
            pltpu.SemaphoreType.DMA,
            pltpu.SemaphoreType.DMA,
        ])
    def scatter_kernel(x_hbm, p0_hbm, p1_hbm, xs_hbm,
                       i0_v, i1_v, rows_v, sem0, sem1):
        wid = jax.lax.axis_index("s") * NC + jax.lax.axis_index("c")
        rbase = wid * NCH
        pltpu.sync_copy(p0_hbm.at[pl.ds(rbase, NCH)], i0_v)
        pltpu.sync_copy(p1_hbm.at[pl.ds(rbase, NCH)], i1_v)

        @pl.loop(0, NCH)
        def _(c):
            base = (rbase + c) * CW
            pltpu.sync_copy(x_hbm.at[pl.ds(base, CW)], rows_v)
            cp0 = pltpu.async_copy(rows_v, xs_hbm.at[i0_v.at[c]], sem0)
            cp1 = pltpu.async_copy(rows_v, xs_hbm.at[i1_v.at[c]], sem1)
            cp0.wait()
            cp1.wait()

    return scatter_kernel(x, p0m, p1m)


def _gateup_body(be_ref, nb_ref, xs_ref, gate_ref, up_ref, hs_ref):
    g = pl.program_id(1)

    @pl.when(g < nb_ref[0])
    def _():
        xs = xs_ref[...].astype(jnp.bfloat16)
        gt = gate_ref[0].astype(jnp.bfloat16)
        ut = up_ref[0].astype(jnp.bfloat16)
        gv = jax.lax.dot_general(xs, gt, (((1,), (1,)), ((), ())),
                                 preferred_element_type=jnp.float32)
        uv = jax.lax.dot_general(xs, ut, (((1,), (1,)), ((), ())),
                                 preferred_element_type=jnp.float32)
        hs_ref[...] = ((gv * jax.lax.logistic(gv)) * uv).astype(jnp.bfloat16)


def _down_body(be_ref, nb_ref, hs_ref, down_ref, ys_ref):
    g = pl.program_id(1)

    @pl.when(g < nb_ref[0] * (T // TB))
    def _():
        dt = down_ref[0].astype(jnp.bfloat16)
        yt = jax.lax.dot_general(dt, hs_ref[...], (((1,), (1,)), ((), ())),
                                 preferred_element_type=jnp.float32)
        ys_ref[...] = yt.T


def _sc_gather(ys, pms):
    mesh = plsc.VectorSubcoreMesh(core_axis_name="c", subcore_axis_name="s")
    otype = jax.ShapeDtypeStruct((N, D), jnp.float32)
    npairs = len(pms) // 2

    @functools.partial(
        pl.kernel,
        out_type=(otype,) * len(pms),
        mesh=mesh,
        scratch_types=(
            [pltpu.VMEM((NCH, CW), jnp.int32)] * len(pms)
            + [pltpu.VMEM((CW, D), jnp.float32)] * 2
            + [pltpu.SemaphoreType.DMA] * 2
        ))
    def gather_kernel(y_hbm, *refs):
        p_hbms = refs[:len(pms)]
        y_hbms = refs[len(pms):2 * len(pms)]
        i_vs = refs[2 * len(pms):3 * len(pms)]
        rows0_v, rows1_v, sem0, sem1 = refs[3 * len(pms):]
        wid = jax.lax.axis_index("s") * NC + jax.lax.axis_index("c")
        rbase = wid * NCH
        for j in range(len(pms)):
            pltpu.sync_copy(p_hbms[j].at[pl.ds(rbase, NCH)], i_vs[j])

        for j in range(npairs):
            ia, ib = i_vs[2 * j], i_vs[2 * j + 1]
            ya, yb = y_hbms[2 * j], y_hbms[2 * j + 1]

            @pl.loop(0, NCH)
            def _(c, ia=ia, ib=ib, ya=ya, yb=yb):
                base = (rbase + c) * CW
                cp0 = pltpu.async_copy(y_hbm.at[ia.at[c]], rows0_v, sem0)
                cp1 = pltpu.async_copy(y_hbm.at[ib.at[c]], rows1_v, sem1)
                cp0.wait()
                cp1.wait()
                pltpu.sync_copy(rows0_v, ya.at[pl.ds(base, CW)])
                pltpu.sync_copy(rows1_v, yb.at[pl.ds(base, CW)])

    return gather_kernel(ys, *pms)


def _combine2_body(y00_ref, y10_ref, w0_ref, w1_ref, out_ref):
    out_ref[...] = w0_ref[...] * y00_ref[...] + w1_ref[...] * y10_ref[...]


def _combine4_body(y00_ref, y10_ref, y01_ref, y11_ref, w0_ref, w1_ref,
                   out_ref):
    out_ref[...] = (w0_ref[...] * (y00_ref[...] + y01_ref[...])
                    + w1_ref[...] * (y10_ref[...] + y11_ref[...]))


def _pipeline(x, router_w, gate_w, up_w, down_w):
    ff = gate_w.shape[1]
    nf = ff // FT
    kc = 2 if ff > 2816 else 1
    fh = ff // kc

    tr = min(1024, N)
    logits = pl.pallas_call(
        _router_body,
        grid=(N // tr,),
        in_specs=[
            pl.BlockSpec((tr, D), lambda t: (t, 0)),
            pl.BlockSpec((NE, D), lambda t: (0, 0)),
        ],
        out_specs=pl.BlockSpec((tr, NE), lambda t: (t, 0)),
        out_shape=jax.ShapeDtypeStruct((N, NE), jnp.float32),
    )(x, router_w)

    pos0, pos1, w0, w1, be, nb = pl.pallas_call(
        _plan_body,
        out_shape=[
            jax.ShapeDtypeStruct((N, 1), jnp.int32),
            jax.ShapeDtypeStruct((N, 1), jnp.int32),
            jax.ShapeDtypeStruct((N, 1), jnp.float32),
            jax.ShapeDtypeStruct((N, 1), jnp.float32),
            jax.ShapeDtypeStruct((1, GB), jnp.int32),
            jax.ShapeDtypeStruct((1, 1), jnp.int32),
        ],
    )(logits)

    p0m = pos0.reshape(N // CW, CW)
    p1m = pos1.reshape(N // CW, CW)
    xs = _sc_scatter(x, p0m, p1m)

    be_flat = be.reshape(GB)
    nb_flat = nb.reshape(1)

    hs = pl.pallas_call(
        _gateup_body,
        grid_spec=pltpu.PrefetchScalarGridSpec(
            num_scalar_prefetch=2,
            grid=(nf, G),
            in_specs=[
                pl.BlockSpec((T, D), lambda f, g, be_r, nb_r: (g, 0)),
                pl.BlockSpec(
                    (1, FT, D),
                    lambda f, g, be_r, nb_r: (be_r[g * (T // TB)], f, 0)),
                pl.BlockSpec(
                    (1, FT, D),
                    lambda f, g, be_r, nb_r: (be_r[g * (T // TB)], f, 0)),
            ],
            out_specs=pl.BlockSpec((T, FT), lambda f, g, be_r, nb_r: (g, f)),
        ),
        out_shape=jax.ShapeDtypeStruct((NS, ff), jnp.bfloat16),
        compiler_params=pltpu.CompilerParams(
            dimension_semantics=("arbitrary", "arbitrary")),
    )(be_flat, nb_flat, xs, gate_w, up_w)

    ys = pl.pallas_call(
        _down_body,
        grid_spec=pltpu.PrefetchScalarGridSpec(
            num_scalar_prefetch=2,
            grid=(kc, GB),
            in_specs=[
                pl.BlockSpec((TB, fh), lambda k, g, be_r, nb_r: (g, k)),
                pl.BlockSpec((1, D, fh),
                             lambda k, g, be_r, nb_r: (be_r[g], 0, k)),
            ],
            out_specs=pl.BlockSpec((TB, D),
                                   lambda k, g, be_r, nb_r: (k * GB + g, 0)),
        ),
        out_shape=jax.ShapeDtypeStruct((kc * NS, D), jnp.float32),
        compiler_params=pltpu.CompilerParams(
            dimension_semantics=("arbitrary", "arbitrary")),
    )(be_flat, nb_flat, hs, down_w)

    if kc == 2:
        ysets = _sc_gather(ys, [p0m, p1m, p0m + NS, p1m + NS])
        cbody = _combine4_body
    else:
        ysets = _sc_gather(ys, [p0m, p1m])
        cbody = _combine2_body

    tc = min(512, N)
    out = pl.pallas_call(
        cbody,
        grid=(N // tc,),
        in_specs=(
            [pl.BlockSpec((tc, D), lambda t: (t, 0))] * len(ysets)
            + [pl.BlockSpec((tc, 1), lambda t: (t, 0))] * 2
        ),
        out_specs=pl.BlockSpec((tc, D), lambda t: (t, 0)),
        out_shape=jax.ShapeDtypeStruct((N, D), jnp.float32),
    )(*ysets, w0, w1)

    return out, logits


def kernel(hidden_states, router_w, gate_w, up_w, down_w):
    b, s, d = hidden_states.shape
    x = hidden_states.reshape(N, D)

    devs = jax.devices()
    if len(devs) >= 2 and devs[0].platform == "tpu":
        import numpy as np
        mesh = jax.sharding.Mesh(np.array(devs[:2]), ("d",))
        pspec = jax.sharding.PartitionSpec

        def fn(x_, rw_, gw_, uw_, dw_):
            o, lg = _pipeline(x_, rw_, gw_, uw_, dw_)
            return jax.lax.psum(o, "d"), lg

        in_specs = (pspec(), pspec(), pspec(None, "d", None),
                    pspec(None, "d", None), pspec(None, None, "d"))
        out_specs = (pspec(), pspec())
        try:
            mapped = jax.shard_map(fn, mesh=mesh, in_specs=in_specs,
                                   out_specs=out_specs, check_vma=False)
        except TypeError:
            mapped = jax.shard_map(fn, mesh=mesh, in_specs=in_specs,
                                   out_specs=out_specs, check_rep=False)
        out, logits = mapped(x, router_w, gate_w, up_w, down_w)
    else:
        out, logits = _pipeline(x, router_w, gate_w, up_w, down_w)

    return out.reshape(b, s, d), logits


# speedup vs baseline: 1.2070x; 1.2070x over previous
import functools

import jax
import jax.numpy as jnp
from jax.experimental import pallas as pl
from jax.experimental.pallas import tpu as pltpu
from jax.experimental.pallas import tpu_sc as plsc

N = 4096
D = 2048
FF = 5632
NE = 8
TOPK = 2
T = 256
G = (N * TOPK) // T + NE
NS = G * T
FT = 1408
NF = FF // FT
TB = 256
GB = NS // TB
KC = 2
FH = FF // KC
NC, NSUB = 2, 16
NW = NC * NSUB
CW = 16
NCH = N // (CW * NW)


def _router_body(x_ref, rw_ref, logits_ref):
    logits_ref[...] = jax.lax.dot_general(
        x_ref[...], rw_ref[...], (((1,), (1,)), ((), ())),
        preferred_element_type=jnp.float32)


def _inclusive_prefix(a):
    n = a.shape[0]
    ck = min(512, n)
    row = jax.lax.broadcasted_iota(jnp.int32, (n, ck), 0)
    acc = None
    for k in range(n // ck):
        colk = jax.lax.broadcasted_iota(jnp.int32, (n, ck), 1) + k * ck
        lt = (colk <= row).astype(jnp.float32)
        part = jax.lax.dot_general(
            lt, a[k * ck:(k + 1) * ck, :], (((1,), (0,)), ((), ())),
            preferred_element_type=jnp.float32)
        acc = part if acc is None else acc + part
    return acc


def _plan_body(logits_ref, pos0_ref, pos1_ref, w0_ref, w1_ref,
               be_ref, nb_ref):
    logits = logits_ref[...]
    m = jnp.max(logits, axis=1, keepdims=True)
    ex = jnp.exp(logits - m)
    p = ex / jnp.sum(ex, axis=1, keepdims=True)
    col = jax.lax.broadcasted_iota(jnp.int32, p.shape, 1)
    m1 = jnp.max(p, axis=1, keepdims=True)
    i1 = jnp.min(jnp.where(p == m1, col, NE), axis=1, keepdims=True)
    oh1 = (col == i1).astype(jnp.float32)
    pmask = jnp.where(col == i1, -jnp.inf, p)
    m2 = jnp.max(pmask, axis=1, keepdims=True)
    i2 = jnp.min(jnp.where(pmask == m2, col, NE), axis=1, keepdims=True)
    oh2 = (col == i2).astype(jnp.float32)
    w0_ref[...] = m1
    w1_ref[...] = m2

    counts = jnp.sum(oh1 + oh2, axis=0, keepdims=True)
    nbpe = jnp.floor((counts + (T - 1)) * (1.0 / T))
    r8 = jax.lax.broadcasted_iota(jnp.int32, (NE, NE), 0)
    c8 = jax.lax.broadcasted_iota(jnp.int32, (NE, NE), 1)
    ltm = (r8 < c8).astype(jnp.float32)
    startsb = jax.lax.dot_general(nbpe, ltm, (((1,), (0,)), ((), ())),
                                  preferred_element_type=jnp.float32)
    starts_tok = startsb * float(T)

    s1i = _inclusive_prefix(oh1)
    s2i = _inclusive_prefix(oh2)
    s1x = s1i - oh1
    s2x = s2i - oh2
    rank_a = jnp.sum(oh1 * (s1x + s2x), axis=1, keepdims=True)
    rank_b = jnp.sum(oh2 * (s1i + s2x), axis=1, keepdims=True)
    start_a = jnp.sum(oh1 * starts_tok, axis=1, keepdims=True)
    start_b = jnp.sum(oh2 * starts_tok, axis=1, keepdims=True)
    pos0_ref[...] = (start_a + rank_a).astype(jnp.int32)
    pos1_ref[...] = (start_b + rank_b).astype(jnp.int32)

    sb_colv = jnp.sum((c8 < r8).astype(jnp.float32) * nbpe, axis=1,
                      keepdims=True) * (float(T) / float(TB))
    gi = jax.lax.broadcasted_iota(jnp.int32, (NE, GB), 1).astype(jnp.float32)
    cmp = (sb_colv <= gi).astype(jnp.float32)
    be_ref[...] = (jnp.sum(cmp, axis=0, keepdims=True) - 1.0).astype(jnp.int32)
    nb_ref[...] = jnp.sum(nbpe, axis=1, keepdims=True).astype(jnp.int32)


def _sc_scatter(x, p0m, p1m):
    mesh = plsc.VectorSubcoreMesh(core_axis_name="c", subcore_axis_name="s")

    @functools.partial(
        pl.kernel,
        out_type=jax.ShapeDtypeStruct((NS, D), jnp.float32),
        mesh=mesh,
        scratch_types=[
            pltpu.VMEM((NCH, CW), jnp.int32),
            pltpu.VMEM((NCH, CW), jnp.int32),
            pltpu.VMEM((CW, D), jnp.float32),
            pltpu.VMEM((CW, D), jnp.float32),
            pltpu.SemaphoreType.DMA,
            pltpu.SemaphoreType.DMA,
            pltpu.SemaphoreType.DMA,
            pltpu.SemaphoreType.DMA,
        ])
    def scatter_kernel(x_hbm, p0_hbm, p1_hbm, xs_hbm,
                       i0_v, i1_v, rows0_v, rows1_v,
                       semla, semlb, sem0, sem1):
        wid = jax.lax.axis_index("s") * NC + jax.lax.axis_index("c")
        rbase = wid * NCH
        pltpu.sync_copy(p0_hbm.at[pl.ds(rbase, NCH)], i0_v)
        pltpu.sync_copy(p1_hbm.at[pl.ds(rbase, NCH)], i1_v)

        pltpu.make_async_copy(
            x_hbm.at[pl.ds(rbase * CW, CW)], rows0_v, semla).start()

        @pl.loop(0, NCH, step=2)
        def _(c):
            base_a = (rbase + c) * CW
            base_b = (rbase + c + 1) * CW
            pltpu.make_async_copy(
                x_hbm.at[pl.ds(base_b, CW)], rows1_v, semlb).start()
            pltpu.make_async_copy(
                x_hbm.at[pl.ds(base_a, CW)], rows0_v, semla).wait()
            s0 = pltpu.make_async_copy(rows0_v, xs_hbm.at[i0_v.at[c]], sem0)
            s1 = pltpu.make_async_copy(rows0_v, xs_hbm.at[i1_v.at[c]], sem1)
            s0.start()
            s1.start()
            s0.wait()
            s1.wait()

            @pl.when(c + 2 < NCH)
            def _():
                pltpu.make_async_copy(
                    x_hbm.at[pl.ds(base_a + 2 * CW, CW)], rows0_v,
                    semla).start()

            pltpu.make_async_copy(
                x_hbm.at[pl.ds(base_b, CW)], rows1_v, semlb).wait()
            t0 = pltpu.make_async_copy(rows1_v, xs_hbm.at[i0_v.at[c + 1]],
                                       sem0)
            t1 = pltpu.make_async_copy(rows1_v, xs_hbm.at[i1_v.at[c + 1]],
                                       sem1)
            t0.start()
            t1.start()
            t0.wait()
            t1.wait()

    return scatter_kernel(x, p0m, p1m)


def _gateup_body(be_ref, nb_ref, xs_ref, gate_ref, up_ref, hs_ref):
    g = pl.program_id(1)

    @pl.when(g < nb_ref[0])
    def _():
        xs = xs_ref[...].astype(jnp.bfloat16)
        gt = gate_ref[0].astype(jnp.bfloat16)
        ut = up_ref[0].astype(jnp.bfloat16)
        gv = jax.lax.dot_general(xs, gt, (((1,), (1,)), ((), ())),
                                 preferred_element_type=jnp.float32)
        uv = jax.lax.dot_general(xs, ut, (((1,), (1,)), ((), ())),
                                 preferred_element_type=jnp.float32)
        hs_ref[...] = ((gv * jax.lax.logistic(gv)) * uv).astype(jnp.bfloat16)


def _down_body(be_ref, nb_ref, hs_ref, down_ref, ys_ref):
    g = pl.program_id(1)

    @pl.when(g < nb_ref[0] * (T // TB))
    def _():
        dt = down_ref[0].astype(jnp.bfloat16)
        yt = jax.lax.dot_general(dt, hs_ref[...], (((1,), (1,)), ((), ())),
                                 preferred_element_type=jnp.float32)
        ys_ref[...] = yt.T


def _sc_gather(ys, p0m, p1m, p0hm, p1hm):
    mesh = plsc.VectorSubcoreMesh(core_axis_name="c", subcore_axis_name="s")
    otype = jax.ShapeDtypeStruct((N, D), jnp.float32)

    @functools.partial(
        pl.kernel,
        out_type=(otype,) * 4,
        mesh=mesh,
        scratch_types=[
            pltpu.VMEM((NCH, CW), jnp.int32),
            pltpu.VMEM((NCH, CW), jnp.int32),
            pltpu.VMEM((NCH, CW), jnp.int32),
            pltpu.VMEM((NCH, CW), jnp.int32),
            pltpu.VMEM((CW, D), jnp.float32),
            pltpu.VMEM((CW, D), jnp.float32),
            pltpu.SemaphoreType.DMA,
            pltpu.SemaphoreType.DMA,
        ])
    def gather_kernel(y_hbm, p0_hbm, p1_hbm, p0h_hbm, p1h_hbm,
                      y00_hbm, y01_hbm, y10_hbm, y11_hbm,
                      i0_v, i1_v, i0h_v, i1h_v, rows0_v, rows1_v,
                      sem0, sem1):
        wid = jax.lax.axis_index("s") * NC + jax.lax.axis_index("c")
        rbase = wid * NCH
        pltpu.sync_copy(p0_hbm.at[pl.ds(rbase, NCH)], i0_v)
        pltpu.sync_copy(p1_hbm.at[pl.ds(rbase, NCH)], i1_v)
        pltpu.sync_copy(p0h_hbm.at[pl.ds(rbase, NCH)], i0h_v)
        pltpu.sync_copy(p1h_hbm.at[pl.ds(rbase, NCH)], i1h_v)

        @pl.loop(0, NCH)
        def _(c):
            base = (rbase + c) * CW
            cp0 = pltpu.async_copy(y_hbm.at[i0_v.at[c]], rows0_v, sem0)
            cp1 = pltpu.async_copy(y_hbm.at[i1_v.at[c]], rows1_v, sem1)
            cp0.wait()
            cp1.wait()
            pltpu.sync_copy(rows0_v, y00_hbm.at[pl.ds(base, CW)])
            pltpu.sync_copy(rows1_v, y10_hbm.at[pl.ds(base, CW)])

        @pl.loop(0, NCH)
        def _(c):
            base = (rbase + c) * CW
            cp0 = pltpu.async_copy(y_hbm.at[i0h_v.at[c]], rows0_v, sem0)
            cp1 = pltpu.async_copy(y_hbm.at[i1h_v.at[c]], rows1_v, sem1)
            cp0.wait()
            cp1.wait()
            pltpu.sync_copy(rows0_v, y01_hbm.at[pl.ds(base, CW)])
            pltpu.sync_copy(rows1_v, y11_hbm.at[pl.ds(base, CW)])

    return gather_kernel(ys, p0m, p1m, p0hm, p1hm)


def _combine_body(y00_ref, y01_ref, y10_ref, y11_ref, w0_ref, w1_ref,
                  out_ref):
    out_ref[...] = (w0_ref[...] * (y00_ref[...] + y01_ref[...])
                    + w1_ref[...] * (y10_ref[...] + y11_ref[...]))


def kernel(hidden_states, router_w, gate_w, up_w, down_w):
    b, s, d = hidden_states.shape
    x = hidden_states.reshape(N, D)

    tr = min(1024, N)
    logits = pl.pallas_call(
        _router_body,
        grid=(N // tr,),
        in_specs=[
            pl.BlockSpec((tr, D), lambda t: (t, 0)),
            pl.BlockSpec((NE, D), lambda t: (0, 0)),
        ],
        out_specs=pl.BlockSpec((tr, NE), lambda t: (t, 0)),
        out_shape=jax.ShapeDtypeStruct((N, NE), jnp.float32),
    )(x, router_w)

    pos0, pos1, w0, w1, be, nb = pl.pallas_call(
        _plan_body,
        out_shape=[
            jax.ShapeDtypeStruct((N, 1), jnp.int32),
            jax.ShapeDtypeStruct((N, 1), jnp.int32),
            jax.ShapeDtypeStruct((N, 1), jnp.float32),
            jax.ShapeDtypeStruct((N, 1), jnp.float32),
            jax.ShapeDtypeStruct((1, GB), jnp.int32),
            jax.ShapeDtypeStruct((1, 1), jnp.int32),
        ],
    )(logits)

    p0m = pos0.reshape(N // CW, CW)
    p1m = pos1.reshape(N // CW, CW)
    xs = _sc_scatter(x, p0m, p1m)

    be_flat = be.reshape(GB)
    nb_flat = nb.reshape(1)

    hs = pl.pallas_call(
        _gateup_body,
        grid_spec=pltpu.PrefetchScalarGridSpec(
            num_scalar_prefetch=2,
            grid=(NF, G),
            in_specs=[
                pl.BlockSpec((T, D), lambda f, g, be_r, nb_r: (g, 0)),
                pl.BlockSpec(
                    (1, FT, D),
                    lambda f, g, be_r, nb_r: (be_r[g * (T // TB)], f, 0)),
                pl.BlockSpec(
                    (1, FT, D),
                    lambda f, g, be_r, nb_r: (be_r[g * (T // TB)], f, 0)),
            ],
            out_specs=pl.BlockSpec((T, FT), lambda f, g, be_r, nb_r: (g, f)),
        ),
        out_shape=jax.ShapeDtypeStruct((NS, FF), jnp.bfloat16),
        compiler_params=pltpu.CompilerParams(
            dimension_semantics=("arbitrary", "arbitrary")),
    )(be_flat, nb_flat, xs, gate_w, up_w)

    ys = pl.pallas_call(
        _down_body,
        grid_spec=pltpu.PrefetchScalarGridSpec(
            num_scalar_prefetch=2,
            grid=(KC, GB),
            in_specs=[
                pl.BlockSpec((TB, FH), lambda k, g, be_r, nb_r: (g, k)),
                pl.BlockSpec((1, D, FH),
                             lambda k, g, be_r, nb_r: (be_r[g], 0, k)),
            ],
            out_specs=pl.BlockSpec((TB, D),
                                   lambda k, g, be_r, nb_r: (k * GB + g, 0)),
        ),
        out_shape=jax.ShapeDtypeStruct((KC * NS, D), jnp.float32),
        compiler_params=pltpu.CompilerParams(
            dimension_semantics=("arbitrary", "arbitrary")),
    )(be_flat, nb_flat, hs, down_w)

    y00, y01, y10, y11 = _sc_gather(ys, p0m, p1m, p0m + NS, p1m + NS)

    tc = min(512, N)
    out = pl.pallas_call(
        _combine_body,
        grid=(N // tc,),
        in_specs=[
            pl.BlockSpec((tc, D), lambda t: (t, 0)),
            pl.BlockSpec((tc, D), lambda t: (t, 0)),
            pl.BlockSpec((tc, D), lambda t: (t, 0)),
            pl.BlockSpec((tc, D), lambda t: (t, 0)),
            pl.BlockSpec((tc, 1), lambda t: (t, 0)),
            pl.BlockSpec((tc, 1), lambda t: (t, 0)),
        ],
        out_specs=pl.BlockSpec((tc, D), lambda t: (t, 0)),
        out_shape=jax.ShapeDtypeStruct((N, D), jnp.float32),
    )(y00, y01, y10, y11, w0, w1)

    return out.reshape(b, s, d), logits


# speedup vs baseline: 1.2101x; 1.0026x over previous
import functools

import jax
import jax.numpy as jnp
from jax.experimental import pallas as pl
from jax.experimental.pallas import tpu as pltpu
from jax.experimental.pallas import tpu_sc as plsc

N = 4096
D = 2048
FF = 5632
NE = 8
TOPK = 2
T = 256
G = (N * TOPK) // T + NE
NS = G * T
FT = 1408
NF = FF // FT
TB = 256
GB = NS // TB
KC = 2
FH = FF // KC
NC, NSUB = 2, 16
NW = NC * NSUB
CW = 16
NCH = N // (CW * NW)


def _router_body(x_ref, rw_ref, logits_ref):
    logits_ref[...] = jax.lax.dot_general(
        x_ref[...], rw_ref[...], (((1,), (1,)), ((), ())),
        preferred_element_type=jnp.float32)


def _inclusive_prefix(a):
    n = a.shape[0]
    ck = min(512, n)
    row = jax.lax.broadcasted_iota(jnp.int32, (n, ck), 0)
    acc = None
    for k in range(n // ck):
        colk = jax.lax.broadcasted_iota(jnp.int32, (n, ck), 1) + k * ck
        lt = (colk <= row).astype(jnp.float32)
        part = jax.lax.dot_general(
            lt, a[k * ck:(k + 1) * ck, :], (((1,), (0,)), ((), ())),
            preferred_element_type=jnp.float32)
        acc = part if acc is None else acc + part
    return acc


def _plan_body(logits_ref, pos0_ref, pos1_ref, w0_ref, w1_ref,
               be_ref, nb_ref):
    logits = logits_ref[...]
    m = jnp.max(logits, axis=1, keepdims=True)
    ex = jnp.exp(logits - m)
    p = ex / jnp.sum(ex, axis=1, keepdims=True)
    col = jax.lax.broadcasted_iota(jnp.int32, p.shape, 1)
    m1 = jnp.max(p, axis=1, keepdims=True)
    i1 = jnp.min(jnp.where(p == m1, col, NE), axis=1, keepdims=True)
    oh1 = (col == i1).astype(jnp.float32)
    pmask = jnp.where(col == i1, -jnp.inf, p)
    m2 = jnp.max(pmask, axis=1, keepdims=True)
    i2 = jnp.min(jnp.where(pmask == m2, col, NE), axis=1, keepdims=True)
    oh2 = (col == i2).astype(jnp.float32)
    w0_ref[...] = m1
    w1_ref[...] = m2

    counts = jnp.sum(oh1 + oh2, axis=0, keepdims=True)
    nbpe = jnp.floor((counts + (T - 1)) * (1.0 / T))
    r8 = jax.lax.broadcasted_iota(jnp.int32, (NE, NE), 0)
    c8 = jax.lax.broadcasted_iota(jnp.int32, (NE, NE), 1)
    ltm = (r8 < c8).astype(jnp.float32)
    startsb = jax.lax.dot_general(nbpe, ltm, (((1,), (0,)), ((), ())),
                                  preferred_element_type=jnp.float32)
    starts_tok = startsb * float(T)

    s1i = _inclusive_prefix(oh1)
    s2i = _inclusive_prefix(oh2)
    s1x = s1i - oh1
    s2x = s2i - oh2
    rank_a = jnp.sum(oh1 * (s1x + s2x), axis=1, keepdims=True)
    rank_b = jnp.sum(oh2 * (s1i + s2x), axis=1, keepdims=True)
    start_a = jnp.sum(oh1 * starts_tok, axis=1, keepdims=True)
    start_b = jnp.sum(oh2 * starts_tok, axis=1, keepdims=True)
    pos0_ref[...] = (start_a + rank_a).astype(jnp.int32)
    pos1_ref[...] = (start_b + rank_b).astype(jnp.int32)

    sb_colv = jnp.sum((c8 < r8).astype(jnp.float32) * nbpe, axis=1,
                      keepdims=True) * (float(T) / float(TB))
    gi = jax.lax.broadcasted_iota(jnp.int32, (NE, GB), 1).astype(jnp.float32)
    cmp = (sb_colv <= gi).astype(jnp.float32)
    be_ref[...] = (jnp.sum(cmp, axis=0, keepdims=True) - 1.0).astype(jnp.int32)
    nb_ref[...] = jnp.sum(nbpe, axis=1, keepdims=True).astype(jnp.int32)


def _sc_scatter(x, p0m, p1m):
    mesh = plsc.VectorSubcoreMesh(core_axis_name="c", subcore_axis_name="s")

    @functools.partial(
        pl.kernel,
        out_type=jax.ShapeDtypeStruct((NS, D), jnp.float32),
        mesh=mesh,
        scratch_types=[
            pltpu.VMEM((NCH, CW), jnp.int32),
            pltpu.VMEM((NCH, CW), jnp.int32),
            pltpu.VMEM((CW, D), jnp.float32),
            pltpu.SemaphoreType.DMA,
            pltpu.SemaphoreType.DMA,
        ])
    def scatter_kernel(x_hbm, p0_hbm, p1_hbm, xs_hbm,
                       i0_v, i1_v, rows_v, sem0, sem1):
        wid = jax.lax.axis_index("s") * NC + jax.lax.axis_index("c")
        rbase = wid * NCH
        pltpu.sync_copy(p0_hbm.at[pl.ds(rbase, NCH)], i0_v)
        pltpu.sync_copy(p1_hbm.at[pl.ds(rbase, NCH)], i1_v)

        @pl.loop(0, NCH)
        def _(c):
            base = (rbase + c) * CW
            pltpu.sync_copy(x_hbm.at[pl.ds(base, CW)], rows_v)
            cp0 = pltpu.async_copy(rows_v, xs_hbm.at[i0_v.at[c]], sem0)
            cp1 = pltpu.async_copy(rows_v, xs_hbm.at[i1_v.at[c]], sem1)
            cp0.wait()
            cp1.wait()

    return scatter_kernel(x, p0m, p1m)


def _gateup_body(be_ref, nb_ref, xs_ref, gate_ref, up_ref, hs_ref):
    g = pl.program_id(1)

    @pl.when(g < nb_ref[0])
    def _():
        xs = xs_ref[...].astype(jnp.bfloat16)
        gt = gate_ref[0].astype(jnp.bfloat16)
        ut = up_ref[0].astype(jnp.bfloat16)
        gv = jax.lax.dot_general(xs, gt, (((1,), (1,)), ((), ())),
                                 preferred_element_type=jnp.float32)
        uv = jax.lax.dot_general(xs, ut, (((1,), (1,)), ((), ())),
                                 preferred_element_type=jnp.float32)
        hs_ref[...] = ((gv * jax.lax.logistic(gv)) * uv).astype(jnp.bfloat16)


def _down_body(be_ref, nb_ref, hs_ref, down_ref, ys_ref):
    g = pl.program_id(1)

    @pl.when(g < nb_ref[0] * (T // TB))
    def _():
        dt = down_ref[0].astype(jnp.bfloat16)
        yt = jax.lax.dot_general(dt, hs_ref[...], (((1,), (1,)), ((), ())),
                                 preferred_element_type=jnp.float32)
        ys_ref[...] = yt.T


def _sc_gather(ys, p0m, p1m, p0hm, p1hm):
    mesh = plsc.VectorSubcoreMesh(core_axis_name="c", subcore_axis_name="s")
    otype = jax.ShapeDtypeStruct((N, D), jnp.float32)

    @functools.partial(
        pl.kernel,
        out_type=(otype,) * 4,
        mesh=mesh,
        scratch_types=[
            pltpu.VMEM((NCH, CW), jnp.int32),
            pltpu.VMEM((NCH, CW), jnp.int32),
            pltpu.VMEM((NCH, CW), jnp.int32),
            pltpu.VMEM((NCH, CW), jnp.int32),
            pltpu.VMEM((CW, D), jnp.float32),
            pltpu.VMEM((CW, D), jnp.float32),
            pltpu.SemaphoreType.DMA,
            pltpu.SemaphoreType.DMA,
        ])
    def gather_kernel(y_hbm, p0_hbm, p1_hbm, p0h_hbm, p1h_hbm,
                      y00_hbm, y01_hbm, y10_hbm, y11_hbm,
                      i0_v, i1_v, i0h_v, i1h_v, rows0_v, rows1_v,
                      sem0, sem1):
        wid = jax.lax.axis_index("s") * NC + jax.lax.axis_index("c")
        rbase = wid * NCH
        pltpu.sync_copy(p0_hbm.at[pl.ds(rbase, NCH)], i0_v)
        pltpu.sync_copy(p1_hbm.at[pl.ds(rbase, NCH)], i1_v)
        pltpu.sync_copy(p0h_hbm.at[pl.ds(rbase, NCH)], i0h_v)
        pltpu.sync_copy(p1h_hbm.at[pl.ds(rbase, NCH)], i1h_v)

        @pl.loop(0, NCH)
        def _(c):
            base = (rbase + c) * CW
            cp0 = pltpu.async_copy(y_hbm.at[i0_v.at[c]], rows0_v, sem0)
            cp1 = pltpu.async_copy(y_hbm.at[i1_v.at[c]], rows1_v, sem1)
            cp0.wait()
            cp1.wait()
            pltpu.sync_copy(rows0_v, y00_hbm.at[pl.ds(base, CW)])
            pltpu.sync_copy(rows1_v, y10_hbm.at[pl.ds(base, CW)])

        @pl.loop(0, NCH)
        def _(c):
            base = (rbase + c) * CW
            cp0 = pltpu.async_copy(y_hbm.at[i0h_v.at[c]], rows0_v, sem0)
            cp1 = pltpu.async_copy(y_hbm.at[i1h_v.at[c]], rows1_v, sem1)
            cp0.wait()
            cp1.wait()
            pltpu.sync_copy(rows0_v, y01_hbm.at[pl.ds(base, CW)])
            pltpu.sync_copy(rows1_v, y11_hbm.at[pl.ds(base, CW)])

    return gather_kernel(ys, p0m, p1m, p0hm, p1hm)


def _combine_body(y00_ref, y01_ref, y10_ref, y11_ref, w0_ref, w1_ref,
                  out_ref):
    out_ref[...] = (w0_ref[...] * (y00_ref[...] + y01_ref[...])
                    + w1_ref[...] * (y10_ref[...] + y11_ref[...]))


def kernel(hidden_states, router_w, gate_w, up_w, down_w):
    b, s, d = hidden_states.shape
    x = hidden_states.reshape(N, D)

    tr = min(1024, N)
    logits = pl.pallas_call(
        _router_body,
        grid=(N // tr,),
        in_specs=[
            pl.BlockSpec((tr, D), lambda t: (t, 0)),
            pl.BlockSpec((NE, D), lambda t: (0, 0)),
        ],
        out_specs=pl.BlockSpec((tr, NE), lambda t: (t, 0)),
        out_shape=jax.ShapeDtypeStruct((N, NE), jnp.float32),
    )(x, router_w)

    pos0, pos1, w0, w1, be, nb = pl.pallas_call(
        _plan_body,
        out_shape=[
            jax.ShapeDtypeStruct((N, 1), jnp.int32),
            jax.ShapeDtypeStruct((N, 1), jnp.int32),
            jax.ShapeDtypeStruct((N, 1), jnp.float32),
            jax.ShapeDtypeStruct((N, 1), jnp.float32),
            jax.ShapeDtypeStruct((1, GB), jnp.int32),
            jax.ShapeDtypeStruct((1, 1), jnp.int32),
        ],
    )(logits)

    p0m = pos0.reshape(N // CW, CW)
    p1m = pos1.reshape(N // CW, CW)
    xs = _sc_scatter(x, p0m, p1m)

    be_flat = be.reshape(GB)
    nb_flat = nb.reshape(1)

    hs = pl.pallas_call(
        _gateup_body,
        grid_spec=pltpu.PrefetchScalarGridSpec(
            num_scalar_prefetch=2,
            grid=(NF, G),
            in_specs=[
                pl.BlockSpec((T, D), lambda f, g, be_r, nb_r: (g, 0)),
                pl.BlockSpec(
                    (1, FT, D),
                    lambda f, g, be_r, nb_r: (be_r[g * (T // TB)], f, 0)),
                pl.BlockSpec(
                    (1, FT, D),
                    lambda f, g, be_r, nb_r: (be_r[g * (T // TB)], f, 0)),
            ],
            out_specs=pl.BlockSpec((T, FT), lambda f, g, be_r, nb_r: (g, f)),
        ),
        out_shape=jax.ShapeDtypeStruct((NS, FF), jnp.bfloat16),
        compiler_params=pltpu.CompilerParams(
            dimension_semantics=("arbitrary", "arbitrary")),
    )(be_flat, nb_flat, xs, gate_w, up_w)

    ys = pl.pallas_call(
        _down_body,
        grid_spec=pltpu.PrefetchScalarGridSpec(
            num_scalar_prefetch=2,
            grid=(KC, GB),
            in_specs=[
                pl.BlockSpec((TB, FH), lambda k, g, be_r, nb_r: (g, k)),
                pl.BlockSpec((1, D, FH),
                             lambda k, g, be_r, nb_r: (be_r[g], 0, k)),
            ],
            out_specs=pl.BlockSpec((TB, D),
                                   lambda k, g, be_r, nb_r: (k * GB + g, 0)),
        ),
        out_shape=jax.ShapeDtypeStruct((KC * NS, D), jnp.float32),
        compiler_params=pltpu.CompilerParams(
            dimension_semantics=("arbitrary", "arbitrary")),
    )(be_flat, nb_flat, hs, down_w)

    y00, y01, y10, y11 = _sc_gather(ys, p0m, p1m, p0m + NS, p1m + NS)

    tc = min(512, N)
    out = pl.pallas_call(
        _combine_body,
        grid=(N // tc,),
        in_specs=[
            pl.BlockSpec((tc, D), lambda t: (t, 0)),
            pl.BlockSpec((tc, D), lambda t: (t, 0)),
            pl.BlockSpec((tc, D), lambda t: (t, 0)),
            pl.BlockSpec((tc, D), lambda t: (t, 0)),
            pl.BlockSpec((tc, 1), lambda t: (t, 0)),
            pl.BlockSpec((tc, 1), lambda t: (t, 0)),
        ],
        out_specs=pl.BlockSpec((tc, D), lambda t: (t, 0)),
        out_shape=jax.ShapeDtypeStruct((N, D), jnp.float32),
    )(y00, y01, y10, y11, w0, w1)

    return out.reshape(b, s, d), logits
